# Initial kernel scaffold; baseline (speedup 1.0000x reference)
import jax
import jax.numpy as jnp
from jax.experimental import pallas as pl


def kernel(edge_index, x_init, edge_type, weight1, root1, bias1, weight2, root2, bias2):
    raise NotImplementedError("write your pallas kernel here")


# speedup vs baseline: 8.1712x; 8.1712x over previous
import functools

import jax
import jax.numpy as jnp
from jax import lax
from jax.experimental import pallas as pl
from jax.experimental.pallas import tpu as pltpu
from jax.experimental.pallas import tpu_sc as plsc

NUM_REL = 4
NC, NS, LANES = 2, 16, 16
NW = NC * NS
B = 128


def _make_scale_kernel(n_pad4, nb_cnt, nb_scl):
    mesh = plsc.VectorSubcoreMesh(core_axis_name="c", subcore_axis_name="s",
                                  num_cores=NC, num_subcores=NS)
    ztile = n_pad4 // NS

    @functools.partial(
        pl.kernel, mesh=mesh,
        out_type=jax.ShapeDtypeStruct((NW, nb_scl, B), jnp.float32),
        scratch_types=[
            pltpu.VMEM((nb_cnt, B), jnp.int32),
            pltpu.VMEM((nb_scl, B), jnp.int32),
            pltpu.VMEM((nb_scl, B), jnp.float32),
            pltpu.VMEM((B,), jnp.float32),
            pltpu.VMEM((B,), jnp.float32),
            pltpu.VMEM((ztile,), jnp.float32),
            pltpu.VMEM_SHARED((n_pad4,), jnp.float32),
            pltpu.SemaphoreType.DMA,
        ],
    )
    def scale_kernel(cidx_scl, scale_out,
                     ci_v, cs_v, sc_v, cb_v, ones_v, z_v, ctab, sem):
        cid = lax.axis_index("c")
        sid = lax.axis_index("s")
        wid = cid * NS + sid

        one16 = jnp.ones((LANES,), jnp.float32)
        zero16 = jnp.zeros((LANES,), jnp.float32)
        for j in range(B // LANES):
            ones_v[pl.ds(j * LANES, LANES)] = one16

        def zfill(i, _):
            z_v[pl.ds(i * LANES, LANES)] = zero16
            return 0
        lax.fori_loop(0, ztile // LANES, zfill, 0)
        pltpu.sync_copy(z_v, ctab.at[pl.ds(sid * ztile, ztile)])
        plsc.subcore_barrier()

        pltpu.sync_copy(cidx_scl.at[2 * sid], ci_v.at[pl.ds(0, nb_cnt // 2)])
        pltpu.sync_copy(cidx_scl.at[2 * sid + 1],
                        ci_v.at[pl.ds(nb_cnt // 2, nb_cnt // 2)])

        def fire(b, _):
            pltpu.async_copy(ones_v, ctab.at[ci_v.at[b]], sem, add=True)
            return 0
        lax.fori_loop(0, nb_cnt, fire, 0)

        def drain(b, _):
            pltpu.make_async_copy(ones_v, ctab.at[ci_v.at[b]], sem).wait()
            return 0
        lax.fori_loop(0, nb_cnt, drain, 0)
        plsc.subcore_barrier()

        pltpu.sync_copy(cidx_scl.at[wid], cs_v)

        def sbody(b, _):
            pltpu.sync_copy(ctab.at[cs_v.at[b]], cb_v)
            for j in range(B // LANES):
                sl = pl.ds(j * LANES, LANES)
                sc_v[b, sl] = 1.0 / jnp.maximum(cb_v[sl], 1.0)
            return 0
        lax.fori_loop(0, nb_scl, sbody, 0)
        pltpu.sync_copy(sc_v, scale_out.at[wid])

    return scale_kernel


def _make_agg_kernel(n_tab, n_acc, nb, d):
    mesh = plsc.VectorSubcoreMesh(core_axis_name="c", subcore_axis_name="s",
                                  num_cores=NC, num_subcores=NS)
    ztile = n_acc // NS

    @functools.partial(
        pl.kernel, mesh=mesh,
        out_type=jax.ShapeDtypeStruct((NC, n_acc, d), jnp.float32),
        scratch_types=[
            pltpu.VMEM((2, B), jnp.int32),
            pltpu.VMEM((2, B), jnp.int32),
            pltpu.VMEM((B,), jnp.float32),
            pltpu.VMEM((B,), jnp.float32),
            pltpu.VMEM((B, d), jnp.float32),
            pltpu.VMEM((B, d), jnp.float32),
            pltpu.VMEM_SHARED((n_acc, d), jnp.float32),
            pltpu.SemaphoreType.DMA,
            pltpu.SemaphoreType.DMA,
            pltpu.SemaphoreType.DMA,
            pltpu.SemaphoreType.DMA,
        ],
    )
    def agg_kernel(tab, rec, s3, znd, parts,
                   eb0, eb1, sb0, sb1, buf0, buf1, acc, es0, es1, gs0, gs1):
        cid = lax.axis_index("c")
        sid = lax.axis_index("s")
        wid = cid * NS + sid

        pltpu.sync_copy(znd.at[pl.ds(sid * ztile, ztile)],
                        acc.at[pl.ds(sid * ztile, ztile)])
        plsc.subcore_barrier()

        def scale_rows(buf, sb):
            def gbody(g, _):
                s16 = sb[pl.ds(g * LANES, LANES)]
                for el in range(LANES):
                    s = s16[el]
                    e = g * LANES + el
                    for j in range(d // LANES):
                        sl = pl.ds(j * LANES, LANES)
                        buf[e, sl] = buf[e, sl] * s
                return 0
            lax.fori_loop(0, B // LANES, gbody, 0)

        ebufs = (eb0, eb1)
        sbufs = (sb0, sb1)
        bufs = (buf0, buf1)
        esems = (es0, es1)
        gsems = (gs0, gs1)

        def start_rec(b, ph):
            pltpu.async_copy(rec.at[wid, b], ebufs[ph], esems[ph])
            pltpu.async_copy(s3.at[wid, b], sbufs[ph], esems[ph])

        def wait_rec(ph):
            pltpu.make_async_copy(rec.at[wid, 0], ebufs[ph], esems[ph]).wait()
            pltpu.make_async_copy(s3.at[wid, 0], sbufs[ph], esems[ph]).wait()

        start_rec(0, 0)
        start_rec(1, 1)
        wait_rec(0)
        pltpu.async_copy(tab.at[eb0.at[0]], buf0, gs0)

        def pair(b2, _):
            for ph in range(2):
                b = 2 * b2 + ph
                eb, sb, buf = ebufs[ph], sbufs[ph], bufs[ph]
                ebn, bufn = ebufs[1 - ph], bufs[1 - ph]
                @pl.when(b < nb - 1)
                def _():
                    wait_rec(1 - ph)
                    pltpu.async_copy(tab.at[ebn.at[0]], bufn, gsems[1 - ph])
                pltpu.make_async_copy(tab.at[eb.at[0]], buf, gsems[ph]).wait()
                scale_rows(buf, sb)
                pltpu.sync_copy(buf, acc.at[eb.at[1]], add=True)
                @pl.when(b < nb - 2)
                def _():
                    start_rec(b + 2, ph)
            return 0
        lax.fori_loop(0, nb // 2, pair, 0)

        plsc.subcore_barrier()
        pltpu.sync_copy(acc.at[pl.ds(sid * ztile, ztile)],
                        parts.at[cid, pl.ds(sid * ztile, ztile)])

    return agg_kernel


def _mm5_body(x_ref, w_ref, b_ref, o_ref):
    r = pl.program_id(0)
    y = jnp.dot(x_ref[...], w_ref[0], preferred_element_type=jnp.float32)
    o_ref[0] = y + jnp.where(r == NUM_REL, 1.0, 0.0) * b_ref[...]


def _mm5_fused_body(base_ref, p_ref, w_ref, b_ref, o_ref):
    r = pl.program_id(0)
    h = jnp.maximum(base_ref[...] + p_ref[0] + p_ref[1], 0.0)
    y = jnp.dot(h, w_ref[0], preferred_element_type=jnp.float32)
    o_ref[0] = y + jnp.where(r == NUM_REL, 1.0, 0.0) * b_ref[...]


def _combine_body(base_ref, p_ref, o_ref):
    o_ref[...] = base_ref[...] + p_ref[0] + p_ref[1]


def _tc_mm5(x, wcat, bias, bn):
    n, d = x.shape
    grid = (NUM_REL + 1, n // bn)
    return pl.pallas_call(
        _mm5_body,
        grid=grid,
        in_specs=[
            pl.BlockSpec((bn, d), lambda r, i: (i, 0)),
            pl.BlockSpec((1, d, d), lambda r, i: (r, 0, 0)),
            pl.BlockSpec((d,), lambda r, i: (0,)),
        ],
        out_specs=pl.BlockSpec((1, bn, d), lambda r, i: (r, i, 0)),
        out_shape=jax.ShapeDtypeStruct((NUM_REL + 1, n, d), jnp.float32),
    )(x, wcat, bias)


def _tc_mm5_fused(base, parts, wcat, bias, bn):
    n, d = base.shape
    grid = (NUM_REL + 1, n // bn)
    return pl.pallas_call(
        _mm5_fused_body,
        grid=grid,
        in_specs=[
            pl.BlockSpec((bn, d), lambda r, i: (i, 0)),
            pl.BlockSpec((NC, bn, d), lambda r, i: (0, i, 0)),
            pl.BlockSpec((1, d, d), lambda r, i: (r, 0, 0)),
            pl.BlockSpec((d,), lambda r, i: (0,)),
        ],
        out_specs=pl.BlockSpec((1, bn, d), lambda r, i: (r, i, 0)),
        out_shape=jax.ShapeDtypeStruct((NUM_REL + 1, n, d), jnp.float32),
    )(base, parts, wcat, bias)


def _tc_combine(base, parts, bn):
    n, d = base.shape
    return pl.pallas_call(
        _combine_body,
        grid=(n // bn,),
        in_specs=[
            pl.BlockSpec((bn, d), lambda i: (i, 0)),
            pl.BlockSpec((NC, bn, d), lambda i: (0, i, 0)),
        ],
        out_specs=pl.BlockSpec((bn, d), lambda i: (i, 0)),
        out_shape=jax.ShapeDtypeStruct((n, d), jnp.float32),
    )(base, parts)


def kernel(edge_index, x_init, edge_type, weight1, root1, bias1,
           weight2, root2, bias2):
    n, d = x_init.shape
    e = edge_index.shape[1]
    src, dst = edge_index[0], edge_index[1]
    et = edge_type

    nb = -(-e // (NW * B))
    if nb % 2:
        nb += 1
    e_pad = NW * nb * B
    nb_cnt = e_pad // (NS * B)
    pad = e_pad - e

    n_acc = -(-(n + 1) // (NS * 8)) * (NS * 8)
    n_tab = NUM_REL * n
    n_pad4 = -(-(n_tab + 1) // (NS * B)) * (NS * B)

    i32 = jnp.int32
    g = (et * n + src).astype(i32)
    cidx = (et * n + dst).astype(i32)
    g_p = jnp.concatenate([g, jnp.zeros((pad,), i32)]).reshape(NW, nb, B)
    dst_p = jnp.concatenate([dst.astype(i32),
                             jnp.full((pad,), n, i32)]).reshape(NW, nb, B)
    cidx_p = jnp.concatenate([cidx, jnp.full((pad,), n_tab, i32)])
    cidx_scl = cidx_p.reshape(NW, nb, B)

    scale3 = _make_scale_kernel(n_pad4, nb_cnt, nb)(cidx_scl)

    rec = jnp.stack([g_p, dst_p], axis=2)

    znd = jnp.zeros((n_acc, d), jnp.float32)
    agg = _make_agg_kernel(n_tab, n_acc, nb, d)

    bn = 2000
    wcat1 = jnp.concatenate([weight1, root1[None]], axis=0)
    wcat2 = jnp.concatenate([weight2, root2[None]], axis=0)

    y1 = _tc_mm5(x_init, wcat1, bias1, bn)
    tab1 = y1[:NUM_REL].reshape(n_tab, d)
    base1 = y1[NUM_REL]
    parts1 = agg(tab1, rec, scale3, znd)[:, :n, :]

    y2 = _tc_mm5_fused(base1, parts1, wcat2, bias2, bn)
    tab2 = y2[:NUM_REL].reshape(n_tab, d)
    base2 = y2[NUM_REL]
    parts2 = agg(tab2, rec, scale3, znd)[:, :n, :]

    return _tc_combine(base2, parts2, bn)


# speedup vs baseline: 8.2351x; 1.0078x over previous
import functools

import jax
import jax.numpy as jnp
from jax import lax
from jax.experimental import pallas as pl
from jax.experimental.pallas import tpu as pltpu
from jax.experimental.pallas import tpu_sc as plsc

NUM_REL = 4
NC, NS, LANES = 2, 16, 16
NW = NC * NS
B = 128


def _make_scale_kernel(n_pad4, nb_cnt, nb_scl):
    mesh = plsc.VectorSubcoreMesh(core_axis_name="c", subcore_axis_name="s",
                                  num_cores=NC, num_subcores=NS)
    ztile = n_pad4 // NS

    @functools.partial(
        pl.kernel, mesh=mesh,
        out_type=jax.ShapeDtypeStruct((NW, nb_scl, B), jnp.float32),
        scratch_types=[
            pltpu.VMEM((nb_cnt, B), jnp.int32),
            pltpu.VMEM((nb_scl, B), jnp.int32),
            pltpu.VMEM((nb_scl, B), jnp.float32),
            pltpu.VMEM((B,), jnp.float32),
            pltpu.VMEM((B,), jnp.float32),
            pltpu.VMEM((ztile,), jnp.float32),
            pltpu.VMEM_SHARED((n_pad4,), jnp.float32),
            pltpu.SemaphoreType.DMA,
        ],
    )
    def scale_kernel(cidx_scl, scale_out,
                     ci_v, cs_v, sc_v, cb_v, ones_v, z_v, ctab, sem):
        cid = lax.axis_index("c")
        sid = lax.axis_index("s")
        wid = cid * NS + sid

        one16 = jnp.ones((LANES,), jnp.float32)
        zero16 = jnp.zeros((LANES,), jnp.float32)
        for j in range(B // LANES):
            ones_v[pl.ds(j * LANES, LANES)] = one16

        def zfill(i, _):
            z_v[pl.ds(i * LANES, LANES)] = zero16
            return 0
        lax.fori_loop(0, ztile // LANES, zfill, 0)
        pltpu.sync_copy(z_v, ctab.at[pl.ds(sid * ztile, ztile)])
        plsc.subcore_barrier()

        pltpu.sync_copy(cidx_scl.at[2 * sid], ci_v.at[pl.ds(0, nb_cnt // 2)])
        pltpu.sync_copy(cidx_scl.at[2 * sid + 1],
                        ci_v.at[pl.ds(nb_cnt // 2, nb_cnt // 2)])

        def fire(b, _):
            pltpu.async_copy(ones_v, ctab.at[ci_v.at[b]], sem, add=True)
            return 0
        lax.fori_loop(0, nb_cnt, fire, 0)

        def drain(b, _):
            pltpu.make_async_copy(ones_v, ctab.at[ci_v.at[b]], sem).wait()
            return 0
        lax.fori_loop(0, nb_cnt, drain, 0)
        plsc.subcore_barrier()

        pltpu.sync_copy(cidx_scl.at[wid], cs_v)

        def sbody(b, _):
            pltpu.sync_copy(ctab.at[cs_v.at[b]], cb_v)
            for j in range(B // LANES):
                sl = pl.ds(j * LANES, LANES)
                sc_v[b, sl] = 1.0 / jnp.maximum(cb_v[sl], 1.0)
            return 0
        lax.fori_loop(0, nb_scl, sbody, 0)
        pltpu.sync_copy(sc_v, scale_out.at[wid])

    return scale_kernel


def _make_agg_kernel(n_tab, n_acc, nb, d):
    mesh = plsc.VectorSubcoreMesh(core_axis_name="c", subcore_axis_name="s",
                                  num_cores=NC, num_subcores=NS)
    ztile = n_acc // NS

    @functools.partial(
        pl.kernel, mesh=mesh,
        out_type=jax.ShapeDtypeStruct((NC, n_acc, d), jnp.float32),
        scratch_types=[
            pltpu.VMEM((B,), jnp.int32),
            pltpu.VMEM((B,), jnp.int32),
            pltpu.VMEM((2, B), jnp.int32),
            pltpu.VMEM((B,), jnp.float32),
            pltpu.VMEM((B,), jnp.float32),
            pltpu.VMEM((B, d), jnp.float32),
            pltpu.VMEM((B, d), jnp.float32),
            pltpu.VMEM_SHARED((n_acc, d), jnp.float32),
        ] + [pltpu.SemaphoreType.DMA] * 8,
    )
    def agg_kernel(tab, g3, d3, s3, znd, parts,
                   gb0, gb1, db, sb0, sb1, buf0, buf1, acc,
                   eg0, eg1, ed0, ed1, gs0, gs1, ss0, ss1):
        cid = lax.axis_index("c")
        sid = lax.axis_index("s")
        wid = cid * NS + sid

        pltpu.sync_copy(znd.at[pl.ds(sid * ztile, ztile)],
                        acc.at[pl.ds(sid * ztile, ztile)])
        plsc.subcore_barrier()

        def scale_rows(buf, sb):
            def gbody(g, _):
                s16 = sb[pl.ds(g * LANES, LANES)]
                for el in range(LANES):
                    s = s16[el]
                    e = g * LANES + el
                    for j in range(d // LANES):
                        sl = pl.ds(j * LANES, LANES)
                        buf[e, sl] = buf[e, sl] * s
                return 0
            lax.fori_loop(0, B // LANES, gbody, 0)

        gbufs, sbufs, bufs = (gb0, gb1), (sb0, sb1), (buf0, buf1)
        egs, eds = (eg0, eg1), (ed0, ed1)
        gsems, ssems = (gs0, gs1), (ss0, ss1)

        pltpu.sync_copy(g3.at[wid, 0], gb0)
        pltpu.async_copy(tab.at[gb0], buf0, gs0)
        pltpu.async_copy(d3.at[wid, 0], db.at[0], ed0)
        pltpu.async_copy(s3.at[wid, 0], sb0, ed0)
        pltpu.async_copy(g3.at[wid, 1], gb1, eg1)

        def pair(k2, _):
            for ph in range(2):
                k = 2 * k2 + ph
                @pl.when(k > 0)
                def _():
                    pltpu.make_async_copy(
                        bufs[1 - ph], acc.at[db.at[1 - ph]],
                        ssems[1 - ph]).wait()
                @pl.when(k < nb - 1)
                def _():
                    pltpu.make_async_copy(g3.at[wid, 0], gbufs[1 - ph],
                                          egs[1 - ph]).wait()
                    pltpu.async_copy(tab.at[gbufs[1 - ph]], bufs[1 - ph],
                                     gsems[1 - ph])
                    pltpu.async_copy(d3.at[wid, k + 1], db.at[1 - ph],
                                     eds[1 - ph])
                    pltpu.async_copy(s3.at[wid, k + 1], sbufs[1 - ph],
                                     eds[1 - ph])
                pltpu.make_async_copy(tab.at[gbufs[ph]], bufs[ph],
                                      gsems[ph]).wait()
                @pl.when(k < nb - 2)
                def _():
                    pltpu.async_copy(g3.at[wid, k + 2], gbufs[ph], egs[ph])
                pltpu.make_async_copy(d3.at[wid, 0], db.at[ph],
                                      eds[ph]).wait()
                pltpu.make_async_copy(s3.at[wid, 0], sbufs[ph], eds[ph]).wait()
                scale_rows(bufs[ph], sbufs[ph])
                pltpu.async_copy(bufs[ph], acc.at[db.at[ph]], ssems[ph],
                                 add=True)
            return 0
        lax.fori_loop(0, nb // 2, pair, 0)

        pltpu.make_async_copy(buf1, acc.at[db.at[1]], ss1).wait()

        plsc.subcore_barrier()
        pltpu.sync_copy(acc.at[pl.ds(sid * ztile, ztile)],
                        parts.at[cid, pl.ds(sid * ztile, ztile)])

    return agg_kernel


def _mm5_body(x_ref, w_ref, b_ref, o_ref):
    r = pl.program_id(0)
    y = jnp.dot(x_ref[...], w_ref[0], preferred_element_type=jnp.float32)
    o_ref[0] = y + jnp.where(r == NUM_REL, 1.0, 0.0) * b_ref[...]


def _mm5_fused_body(base_ref, p_ref, w_ref, b_ref, o_ref):
    r = pl.program_id(0)
    h = jnp.maximum(base_ref[...] + p_ref[0] + p_ref[1], 0.0)
    y = jnp.dot(h, w_ref[0], preferred_element_type=jnp.float32)
    o_ref[0] = y + jnp.where(r == NUM_REL, 1.0, 0.0) * b_ref[...]


def _combine_body(base_ref, p_ref, o_ref):
    o_ref[...] = base_ref[...] + p_ref[0] + p_ref[1]


def _tc_mm5(x, wcat, bias, bn):
    n, d = x.shape
    grid = (NUM_REL + 1, n // bn)
    return pl.pallas_call(
        _mm5_body,
        grid=grid,
        in_specs=[
            pl.BlockSpec((bn, d), lambda r, i: (i, 0)),
            pl.BlockSpec((1, d, d), lambda r, i: (r, 0, 0)),
            pl.BlockSpec((d,), lambda r, i: (0,)),
        ],
        out_specs=pl.BlockSpec((1, bn, d), lambda r, i: (r, i, 0)),
        out_shape=jax.ShapeDtypeStruct((NUM_REL + 1, n, d), jnp.float32),
    )(x, wcat, bias)


def _tc_mm5_fused(base, parts, wcat, bias, bn):
    n, d = base.shape
    grid = (NUM_REL + 1, n // bn)
    return pl.pallas_call(
        _mm5_fused_body,
        grid=grid,
        in_specs=[
            pl.BlockSpec((bn, d), lambda r, i: (i, 0)),
            pl.BlockSpec((NC, bn, d), lambda r, i: (0, i, 0)),
            pl.BlockSpec((1, d, d), lambda r, i: (r, 0, 0)),
            pl.BlockSpec((d,), lambda r, i: (0,)),
        ],
        out_specs=pl.BlockSpec((1, bn, d), lambda r, i: (r, i, 0)),
        out_shape=jax.ShapeDtypeStruct((NUM_REL + 1, n, d), jnp.float32),
    )(base, parts, wcat, bias)


def _tc_combine(base, parts, bn):
    n, d = base.shape
    return pl.pallas_call(
        _combine_body,
        grid=(n // bn,),
        in_specs=[
            pl.BlockSpec((bn, d), lambda i: (i, 0)),
            pl.BlockSpec((NC, bn, d), lambda i: (0, i, 0)),
        ],
        out_specs=pl.BlockSpec((bn, d), lambda i: (i, 0)),
        out_shape=jax.ShapeDtypeStruct((n, d), jnp.float32),
    )(base, parts)


def kernel(edge_index, x_init, edge_type, weight1, root1, bias1,
           weight2, root2, bias2):
    n, d = x_init.shape
    e = edge_index.shape[1]
    src, dst = edge_index[0], edge_index[1]
    et = edge_type

    nb = -(-e // (NW * B))
    if nb % 2:
        nb += 1
    e_pad = NW * nb * B
    nb_cnt = e_pad // (NS * B)
    pad = e_pad - e

    n_acc = -(-(n + 1) // (NS * 8)) * (NS * 8)
    n_tab = NUM_REL * n
    n_pad4 = -(-(n_tab + 1) // (NS * B)) * (NS * B)

    i32 = jnp.int32
    g = (et * n + src).astype(i32)
    cidx = (et * n + dst).astype(i32)
    g_p = jnp.concatenate([g, jnp.zeros((pad,), i32)]).reshape(NW, nb, B)
    dst_p = jnp.concatenate([dst.astype(i32),
                             jnp.full((pad,), n, i32)]).reshape(NW, nb, B)
    cidx_p = jnp.concatenate([cidx, jnp.full((pad,), n_tab, i32)])
    cidx_scl = cidx_p.reshape(NW, nb, B)

    scale3 = _make_scale_kernel(n_pad4, nb_cnt, nb)(cidx_scl)

    znd = jnp.zeros((n_acc, d), jnp.float32)
    agg = _make_agg_kernel(n_tab, n_acc, nb, d)

    bn = 2000
    wcat1 = jnp.concatenate([weight1, root1[None]], axis=0)
    wcat2 = jnp.concatenate([weight2, root2[None]], axis=0)

    y1 = _tc_mm5(x_init, wcat1, bias1, bn)
    tab1 = y1[:NUM_REL].reshape(n_tab, d)
    base1 = y1[NUM_REL]
    parts1 = agg(tab1, g_p, dst_p, scale3, znd)[:, :n, :]

    y2 = _tc_mm5_fused(base1, parts1, wcat2, bias2, bn)
    tab2 = y2[:NUM_REL].reshape(n_tab, d)
    base2 = y2[NUM_REL]
    parts2 = agg(tab2, g_p, dst_p, scale3, znd)[:, :n, :]

    return _tc_combine(base2, parts2, bn)
